# speedup vs baseline: 1.3301x; 1.3301x over previous
import functools

import jax
import jax.numpy as jnp
from jax import lax
from jax.experimental import pallas as pl
from jax.experimental.pallas import tpu as pltpu
from jax.experimental.pallas import tpu_sc as plsc

B, T, E = 1024, 512, 64
D = E - 1
ZLEN = D * E
NC, NS = 2, 16
NW = NC * NS
NB = B // NW
CH = 8
NCHUNK = NB // CH
TAIL_OFF = (T - D) * E
SAMPLE = T * E

_mesh = plsc.VectorSubcoreMesh(core_axis_name="c", subcore_axis_name="s")


@functools.partial(
    pl.kernel,
    out_type=jax.ShapeDtypeStruct((B * D,), jnp.float32),
    mesh=_mesh,
    scratch_types=[
        pltpu.VMEM((CH * ZLEN + 8,), jnp.float32),
        pltpu.VMEM((CH * ZLEN + 8,), jnp.float32),
        pltpu.VMEM((NB * D + 8,), jnp.float32),
        pltpu.SemaphoreType.DMA,
        pltpu.SemaphoreType.DMA,
    ],
)
def _anti_diag_means(in_hbm, out_hbm, buf0, buf1, obuf, sem0, sem1):
    wid = lax.axis_index("s") * NC + lax.axis_index("c")
    b0 = wid * NB

    iota = lax.broadcasted_iota(jnp.int32, (16,), 0)
    masks = [iota <= m for m in range(16)]
    rcp = [1.0 / (D - 16 * q - iota).astype(jnp.float32) for q in range(4)]

    def fire(chunk, buf, sem):
        cps = []
        for jj in range(CH):
            b = b0 + chunk * CH + jj
            src = in_hbm.at[pl.ds(b * SAMPLE + TAIL_OFF, ZLEN)]
            dst = buf.at[pl.ds(jj * ZLEN, ZLEN)]
            cps.append(pltpu.async_copy(src, dst, sem))
        return cps

    def compute(buf, obase):
        def batch_body(jj, carry):
            zb = jj * ZLEN
            acc = [jnp.zeros((16,), jnp.float32) for _ in range(4)]
            for r in range(D):
                base = zb + 63 * r + 63
                qmax = r // 16
                for q in range(qmax + 1):
                    v = buf[pl.ds(base + 16 * q, 16)]
                    if q == qmax:
                        v = jnp.where(masks[r % 16], v, 0.0)
                    acc[q] = acc[q] + v
            ob = obase + jj * D
            for q in range(4):
                obuf[pl.ds(ob + 16 * q, 16)] = acc[q] * rcp[q]
            return carry
        lax.fori_loop(0, CH, batch_body, 0)

    cps = fire(0, buf0, sem0)
    for chunk in range(NCHUNK):
        cur_buf = buf0 if chunk % 2 == 0 else buf1
        cur_cps = cps
        if chunk + 1 < NCHUNK:
            cps = fire(chunk + 1,
                       buf1 if chunk % 2 == 0 else buf0,
                       sem1 if chunk % 2 == 0 else sem0)
        for cp in cur_cps:
            cp.wait()
        compute(cur_buf, chunk * CH * D)

    pltpu.sync_copy(obuf.at[pl.ds(0, NB * D)],
                    out_hbm.at[pl.ds(wid * NB * D, NB * D)])


def kernel(inputs):
    flat = inputs.reshape(-1)
    out = _anti_diag_means(flat)
    return out.reshape(B, D)


# speedup vs baseline: 4.5884x; 3.4497x over previous
import functools

import jax
import jax.numpy as jnp
from jax import lax
from jax.experimental import pallas as pl
from jax.experimental.pallas import tpu as pltpu
from jax.experimental.pallas import tpu_sc as plsc

B, T, E = 1024, 512, 64
D = E - 1
ZLEN = E * E
NC, NS = 2, 16
NW = NC * NS
NB = B // NW
CH = 8
NCHUNK = NB // CH

_mesh = plsc.VectorSubcoreMesh(core_axis_name="c", subcore_axis_name="s")


@functools.partial(
    pl.kernel,
    out_type=jax.ShapeDtypeStruct((B * D,), jnp.float32),
    mesh=_mesh,
    scratch_types=[
        pltpu.VMEM((CH * ZLEN + 8,), jnp.float32),
        pltpu.VMEM((CH * ZLEN + 8,), jnp.float32),
        pltpu.VMEM((NB * D + 8,), jnp.float32),
        pltpu.SemaphoreType.DMA,
        pltpu.SemaphoreType.DMA,
    ],
)
def _anti_diag_means(in_hbm, out_hbm, buf0, buf1, obuf, sem0, sem1):
    wid = lax.axis_index("s") * NC + lax.axis_index("c")
    b0 = wid * NB

    iota = lax.broadcasted_iota(jnp.int32, (16,), 0)
    masks = [iota <= m for m in range(16)]
    rcp = [1.0 / (D - 16 * q - iota).astype(jnp.float32) for q in range(4)]

    def fire(chunk, buf, sem):
        cps = []
        for jj in range(CH):
            b = b0 + chunk * CH + jj
            src = in_hbm.at[pl.ds(b * ZLEN, ZLEN)]
            dst = buf.at[pl.ds(jj * ZLEN, ZLEN)]
            cps.append(pltpu.async_copy(src, dst, sem))
        return cps

    def compute(buf, obase):
        def batch_body(jj, carry):
            zb = jj * ZLEN
            acc = [jnp.zeros((16,), jnp.float32) for _ in range(4)]
            for r in range(D):
                base = zb + 63 * r + 127
                qmax = r // 16
                for q in range(qmax + 1):
                    v = buf[pl.ds(base + 16 * q, 16)]
                    if q == qmax:
                        v = jnp.where(masks[r % 16], v, 0.0)
                    acc[q] = acc[q] + v
            ob = obase + jj * D
            for q in range(4):
                obuf[pl.ds(ob + 16 * q, 16)] = acc[q] * rcp[q]
            return carry
        lax.fori_loop(0, CH, batch_body, 0)

    cps = fire(0, buf0, sem0)
    for chunk in range(NCHUNK):
        cur_buf = buf0 if chunk % 2 == 0 else buf1
        cur_cps = cps
        if chunk + 1 < NCHUNK:
            cps = fire(chunk + 1,
                       buf1 if chunk % 2 == 0 else buf0,
                       sem1 if chunk % 2 == 0 else sem0)
        for cp in cur_cps:
            cp.wait()
        compute(cur_buf, chunk * CH * D)

    pltpu.sync_copy(obuf.at[pl.ds(0, NB * D)],
                    out_hbm.at[pl.ds(wid * NB * D, NB * D)])


def kernel(inputs):
    tail = inputs[:, T - E:, :].reshape(-1)
    out = _anti_diag_means(tail)
    return out.reshape(B, D)
